# baseline (speedup 1.0000x reference)
import functools

import jax
import jax.numpy as jnp
from jax.experimental import pallas as pl
from jax.experimental.pallas import tpu as pltpu

_ALPHA = 0.2
_EB = 512


def _lrelu(v):
    return jnp.maximum(v, _ALPHA * v)


def _bf(v):
    return v.astype(jnp.bfloat16).astype(jnp.float32)


def _stage1_body(R, C, xt_ref, mt_ref, hnb_ref, cst_ref, inc_ref,
                 out1_ref, sout_ref, s_scr):
    ge = pl.program_id(0)

    @pl.when(ge == 0)
    def _compute_s():
        wx = cst_ref[:, 0:1]
        wm = cst_ref[:, 1:2]
        w_pn = cst_ref[0:1, 2:3]
        for r in range(R):
            sr = (jnp.sum(_bf(xt_ref[r]) * wx, axis=0, keepdims=True)
                  + jnp.sum(_bf(mt_ref[r]) * wm, axis=0, keepdims=True)
                  + hnb_ref[0:1, :] * w_pn
                  + hnb_ref[1:2, :])
            s_scr[r:r + 1, :] = _bf(sr)
        s_scr[R:R + 1, :] = jnp.ones_like(s_scr[R:R + 1, :])
        if R + 1 < 8:
            s_scr[R + 1:, :] = jnp.zeros_like(s_scr[R + 1:, :])
        sout_ref[:, :] = s_scr[:, :]

    mm = jnp.dot(s_scr[:, :], inc_ref[:, :],
                 preferred_element_type=jnp.float32)
    out1_ref[:, :] = mm * (1.0 / mm[R:R + 1, :])


def _stage2_body(R, ep_ref, st_ref, cst_ref, w2_ref, inc_ref,
                 edge_ref, node_ref, mx_scr):
    ge = pl.program_id(0)
    a0L = cst_ref[1:2, 2:3]
    a1L = cst_ref[2:3, 2:3]

    edge_all = jnp.dot(ep_ref[:, :], w2_ref[:, :],
                       preferred_element_type=jnp.float32)
    edge_ref[:, :] = edge_all

    @pl.when(ge == 0)
    def _init():
        node_ref[:, :] = jnp.zeros_like(node_ref)
        mx_scr[:, :] = jnp.zeros_like(mx_scr)
        for r in range(R):
            mx_scr[r:r + 1, 0:1] = jnp.max(st_ref[:, r:r + 1] * a0L,
                                           axis=0, keepdims=True)

    inc = inc_ref[:, :]
    iota8 = jax.lax.broadcasted_iota(jnp.int32, (8, 1), 0)
    acc = None
    for r in range(R):
        vq = st_ref[:, r:r + 1] * a0L
        vy = _bf(edge_all[r:r + 1, :]) * a1L
        mr = _lrelu(mx_scr[r:r + 1, 0:1] + vy)
        attn_un = inc * jnp.exp2(_lrelu(vq + vy) - mr)
        dr = jnp.sum(attn_un, axis=0, keepdims=True)
        wrows = jnp.where(iota8 == r, edge_all[r:r + 1, :] / dr,
                          0.0)
        part = jax.lax.dot_general(
            attn_un, wrows, (((1,), (1,)), ((), ())),
            precision=jax.lax.Precision.HIGHEST,
            preferred_element_type=jnp.float32)
        acc = part if acc is None else acc + part
    node_ref[:, :] += acc


def kernel(x, incidence, mask, h_node, h_e, weight, bias, weight2, a):
    B, C, N, T = x.shape
    E = incidence.shape[1]
    R = B * T
    G = E // _EB

    f32 = jnp.float32
    bf = lambda v: v.astype(jnp.bfloat16).astype(f32)
    L2E = 1.4426950408889634
    xt = jnp.transpose(x, (0, 3, 1, 2)).reshape(R, C, N)
    mt = jnp.transpose(mask, (0, 3, 1, 2)).reshape(R, C, N)
    hnb = jnp.concatenate(
        [bf(h_node), bias[None, :], jnp.zeros((6, N), f32)], axis=0)
    cst = (jnp.zeros((8, 128), f32)
           .at[:C, 0].set(bf(weight[:C, 0]))
           .at[:C, 1].set(bf(weight[C:2 * C, 0]))
           .at[0, 2].set(bf(weight[2 * C, 0]))
           .at[1, 2].set(bf(a[0, 0]) * L2E)
           .at[2, 2].set(bf(a[1, 0]) * L2E))

    ep, s_pad = pl.pallas_call(
        functools.partial(_stage1_body, R, C),
        grid=(G,),
        in_specs=[
            pl.BlockSpec((R, C, N), lambda i: (0, 0, 0)),
            pl.BlockSpec((R, C, N), lambda i: (0, 0, 0)),
            pl.BlockSpec((8, N), lambda i: (0, 0)),
            pl.BlockSpec((8, 128), lambda i: (0, 0)),
            pl.BlockSpec((N, _EB), lambda i: (0, i)),
        ],
        out_specs=[
            pl.BlockSpec((8, _EB), lambda i: (0, i)),
            pl.BlockSpec((8, N), lambda i: (0, 0)),
        ],
        out_shape=[
            jax.ShapeDtypeStruct((8, E), f32),
            jax.ShapeDtypeStruct((8, N), f32),
        ],
        scratch_shapes=[pltpu.VMEM((8, N), f32)],
    )(xt, mt, hnb, cst, incidence)

    st = s_pad.T

    edge_out, node_t = pl.pallas_call(
        functools.partial(_stage2_body, R),
        grid=(G,),
        in_specs=[
            pl.BlockSpec((8, E), lambda i: (0, 0)),
            pl.BlockSpec((N, 8), lambda i: (0, 0)),
            pl.BlockSpec((8, 128), lambda i: (0, 0)),
            pl.BlockSpec((E, _EB), lambda i: (0, i)),
            pl.BlockSpec((N, _EB), lambda i: (0, i)),
        ],
        out_specs=[
            pl.BlockSpec((8, _EB), lambda i: (0, i)),
            pl.BlockSpec((N, 8), lambda i: (0, 0)),
        ],
        out_shape=[
            jax.ShapeDtypeStruct((8, E), f32),
            jax.ShapeDtypeStruct((N, 8), f32),
        ],
        scratch_shapes=[pltpu.VMEM((8, 128), f32)],
    )(ep, st, cst, weight2, incidence)

    imputations = jnp.transpose(
        node_t[:, :R].reshape(N, B, T), (1, 0, 2))[:, None, :, :]
    edge_last = edge_out[T - 1:R:T, :][:, :, None]
    return imputations, edge_last


# speedup vs baseline: 1.6570x; 1.6570x over previous
import functools

import jax
import jax.numpy as jnp
from jax.experimental import pallas as pl
from jax.experimental.pallas import tpu as pltpu

_ALPHA = 0.2
_EB = 512


def _lrelu(v):
    return jnp.maximum(v, _ALPHA * v)


def _bf(v):
    return v.astype(jnp.bfloat16).astype(jnp.float32)


def _body(R, C, G, xt_ref, mt_ref, hnb_ref, cst_ref, inc_ref, w2_ref,
          edge_ref, node_ref, s_scr, st_scr, ep_scr, mx_scr):
    ge = pl.program_id(0)
    a0L = cst_ref[1:2, 2:3]
    a1L = cst_ref[2:3, 2:3]

    @pl.when(ge == 0)
    def _compute_s():
        wx = cst_ref[:, 0:1]
        wm = cst_ref[:, 1:2]
        w_pn = cst_ref[0:1, 2:3]
        for r in range(R):
            sr = (jnp.sum(_bf(xt_ref[r]) * wx, axis=0, keepdims=True)
                  + jnp.sum(_bf(mt_ref[r]) * wm, axis=0, keepdims=True)
                  + hnb_ref[0:1, :] * w_pn
                  + hnb_ref[1:2, :])
            s_scr[r:r + 1, :] = _bf(sr)
        s_scr[R:R + 1, :] = jnp.ones_like(s_scr[R:R + 1, :])
        if R + 1 < 8:
            s_scr[R + 1:, :] = jnp.zeros_like(s_scr[R + 1:, :])
        st_scr[:, :] = jnp.transpose(s_scr[:, :], (1, 0))
        for r in range(R):
            mx_scr[r:r + 1, 0:1] = jnp.max(st_scr[:, r:r + 1] * a0L,
                                           axis=0, keepdims=True)
        node_ref[:, :] = jnp.zeros_like(node_ref)

    @pl.when(ge < G)
    def _phase1():
        mm = jnp.dot(s_scr[:, :], inc_ref[:, :],
                     preferred_element_type=jnp.float32)
        ep_scr[:, pl.ds(ge * _EB, _EB)] = mm * (1.0 / mm[R:R + 1, :])

    @pl.when(ge >= G)
    def _phase2():
        edge_all = jnp.dot(ep_scr[:, :], w2_ref[:, :],
                           preferred_element_type=jnp.float32)
        edge_ref[:, :] = edge_all
        inc = inc_ref[:, :]
        ones_row = jnp.ones((8, inc.shape[0]), jnp.float32)
        iota8 = jax.lax.broadcasted_iota(jnp.int32, (8, 1), 0)
        acc = None
        for r in range(R):
            vq = st_scr[:, r:r + 1] * a0L
            vy = _bf(edge_all[r:r + 1, :]) * a1L
            mr = _lrelu(mx_scr[r:r + 1, 0:1] + vy)
            attn_un = inc * jnp.exp2(_lrelu(vq + vy) - mr)
            dr = jnp.dot(ones_row, attn_un,
                         preferred_element_type=jnp.float32)[0:1]
            wrows = jnp.where(iota8 == r, edge_all[r:r + 1, :] / dr,
                              0.0)
            part = jax.lax.dot_general(
                attn_un, wrows, (((1,), (1,)), ((), ())),
                preferred_element_type=jnp.float32)
            acc = part if acc is None else acc + part
        node_ref[:, :] += acc


def kernel(x, incidence, mask, h_node, h_e, weight, bias, weight2, a):
    B, C, N, T = x.shape
    E = incidence.shape[1]
    R = B * T
    G = E // _EB

    f32 = jnp.float32
    bf = lambda v: v.astype(jnp.bfloat16).astype(f32)
    L2E = 1.4426950408889634
    xt = jnp.transpose(x, (0, 3, 1, 2)).reshape(R, C, N)
    mt = jnp.transpose(mask, (0, 3, 1, 2)).reshape(R, C, N)
    hnb = jnp.concatenate(
        [bf(h_node), bias[None, :], jnp.zeros((6, N), f32)], axis=0)
    cst = (jnp.zeros((8, 128), f32)
           .at[:C, 0].set(bf(weight[:C, 0]))
           .at[:C, 1].set(bf(weight[C:2 * C, 0]))
           .at[0, 2].set(bf(weight[2 * C, 0]))
           .at[1, 2].set(bf(a[0, 0]) * L2E)
           .at[2, 2].set(bf(a[1, 0]) * L2E))

    edge_out, node_t = pl.pallas_call(
        functools.partial(_body, R, C, G),
        grid=(2 * G,),
        in_specs=[
            pl.BlockSpec((R, C, N), lambda i: (0, 0, 0)),
            pl.BlockSpec((R, C, N), lambda i: (0, 0, 0)),
            pl.BlockSpec((8, N), lambda i: (0, 0)),
            pl.BlockSpec((8, 128), lambda i: (0, 0)),
            pl.BlockSpec((N, _EB), lambda i: (0, i % G)),
            pl.BlockSpec((E, _EB), lambda i: (0, jnp.maximum(i - G, 0))),
        ],
        out_specs=[
            pl.BlockSpec((8, _EB), lambda i: (0, jnp.maximum(i - G, 0))),
            pl.BlockSpec((N, 8), lambda i: (0, 0)),
        ],
        out_shape=[
            jax.ShapeDtypeStruct((8, E), f32),
            jax.ShapeDtypeStruct((N, 8), f32),
        ],
        scratch_shapes=[
            pltpu.VMEM((8, N), f32),
            pltpu.VMEM((N, 8), f32),
            pltpu.VMEM((8, E), f32),
            pltpu.VMEM((8, 128), f32),
        ],
    )(xt, mt, hnb, cst, incidence, weight2)

    imputations = jnp.transpose(
        node_t[:, :R].reshape(N, B, T), (1, 0, 2))[:, None, :, :]
    edge_last = edge_out[T - 1:R:T, :][:, :, None]
    return imputations, edge_last
